# baseline (speedup 1.0000x reference)
import functools

import jax
import jax.numpy as jnp
from jax import lax
from jax.experimental import pallas as pl
from jax.experimental.pallas import tpu as pltpu
from jax.experimental.pallas import tpu_sc as plsc

B = 16384
F = 26
V = 100000
D = 16

_INFO = plsc.get_sparse_core_info()
NC = _INFO.num_cores
NS = _INFO.num_subcores
L = _INFO.num_lanes
NW = NC * NS

N = B * F
PW = N // NW
C = 3328
NCH = PW // C

_mesh = plsc.VectorSubcoreMesh(core_axis_name="c", subcore_axis_name="s")


@functools.partial(
    pl.kernel,
    mesh=_mesh,
    out_type=jax.ShapeDtypeStruct((N, D), jnp.float32),
    scratch_types=[
        pltpu.VMEM((C,), jnp.int32),
        pltpu.VMEM((C, D), jnp.float32),
        pltpu.SemaphoreType.DMA,
    ],
    compiler_params=pltpu.CompilerParams(use_tc_tiling_on_sc=False),
)
def _gather_rows(table_hbm, idx_hbm, out_hbm, idx_v, rows_v, sem):
    wid = lax.axis_index("s") * NC + lax.axis_index("c")
    base = wid * PW

    def chunk_body(i, _):
        off = base + i * C
        pltpu.sync_copy(idx_hbm.at[pl.ds(off, C)], idx_v)

        def add_off(j, _):
            pos = lax.iota(jnp.int32, L) + (off + j * L)
            f_col = pos % F
            idx_v[pl.ds(j * L, L)] = idx_v[pl.ds(j * L, L)] + f_col * V
            return ()

        lax.fori_loop(0, C // L, add_off, ())

        pltpu.async_copy(table_hbm.at[idx_v], rows_v, sem).wait()

        pltpu.sync_copy(rows_v, out_hbm.at[pl.ds(off, C)])
        return ()

    lax.fori_loop(0, NCH, chunk_body, ())


def kernel(indices, tables):
    table_2d = tables.reshape(F * V, D)
    idx_flat = indices.reshape(N)
    out = _gather_rows(table_2d, idx_flat)
    return out.reshape(B, F * D)
